# baseline (speedup 1.0000x reference)
import functools

import jax
import jax.numpy as jnp
from jax.experimental import pallas as pl
from jax.experimental.pallas import tpu as pltpu

TOPK = 5
_BIG = 2**30


def _retrieve_kernel(q_ref, *rest, ns, blk, seg, topk):
    k_refs = rest[:ns]
    n_refs = rest[ns:2 * ns]
    out_v_ref, out_i_ref = rest[2 * ns], rest[2 * ns + 1]
    s_scr, cand_v, cand_i = rest[2 * ns + 2], rest[2 * ns + 3], rest[2 * ns + 4]
    i = pl.program_id(0)

    @pl.when(i == 0)
    def _init():
        out_v_ref[...] = jnp.full(out_v_ref.shape, -jnp.inf, jnp.float32)
        out_i_ref[...] = jnp.zeros(out_i_ref.shape, jnp.int32)

    parts = []
    for k_ref, n_ref in zip(k_refs, n_refs):
        kn = k_ref[...] / (n_ref[...] + 1e-8)
        parts.append(jax.lax.dot_general(
            q_ref[...], kn, (((1,), (1,)), ((), ())),
            preferred_element_type=jnp.float32,
        ))
    s = jnp.concatenate(parts, axis=1) if ns > 1 else parts[0]

    iota_b = jax.lax.broadcasted_iota(jnp.int32, s.shape, 1)
    gidx = iota_b + i * blk + (iota_b // blk) * (seg - blk)

    thresh = out_v_ref[:, topk - 1][:, None]
    rowcnt = jnp.sum((s > thresh).astype(jnp.int32), axis=1, keepdims=True)
    mc = jnp.max(rowcnt)

    @pl.when(mc > 0)
    def _update():
        s_scr[...] = s
        cand_v[...] = jnp.full(cand_v.shape, -jnp.inf, jnp.float32)
        cand_i[...] = jnp.zeros(cand_i.shape, jnp.int32)

        for j in range(topk):
            @pl.when(j < mc)
            def _extract(j=j):
                sv = s_scr[...]
                m = jnp.max(sv, axis=1)
                idxc = jnp.where(sv == m[:, None], gidx, _BIG)
                a = jnp.min(idxc, axis=1)
                cand_v[:, j:j + 1] = m[:, None]
                cand_i[:, j:j + 1] = a[:, None]
                s_scr[...] = jnp.where(idxc == a[:, None], -jnp.inf, sv)

        cv = jnp.concatenate([out_v_ref[...], cand_v[...]], axis=1)
        ci = jnp.concatenate([out_i_ref[...], cand_i[...]], axis=1)
        nv_cols, ni_cols = [], []
        for _ in range(topk):
            m = jnp.max(cv, axis=1)
            idxc = jnp.where(cv == m[:, None], ci, _BIG)
            a = jnp.min(idxc, axis=1)
            oh = (cv == m[:, None]) & (ci == a[:, None])
            nv_cols.append(m[:, None])
            ni_cols.append(a[:, None])
            cv = jnp.where(oh, -jnp.inf, cv)
        out_v_ref[...] = jnp.concatenate(nv_cols, axis=1)
        out_i_ref[...] = jnp.concatenate(ni_cols, axis=1)


def _retrieve(qn, keys, knorm, ns, blk, interpret=False):
    Q, D = qn.shape
    K, _ = keys.shape
    seg = K // ns
    assert seg % blk == 0 and seg * ns == K
    nblk = seg // blk
    kfn = functools.partial(_retrieve_kernel, ns=ns, blk=blk, seg=seg,
                            topk=TOPK)
    kspecs = [
        pl.BlockSpec((blk, D), lambda i, j=j: (i + j * nblk, 0))
        for j in range(ns)
    ]
    nspecs = [
        pl.BlockSpec((blk, 1), lambda i, j=j: (i + j * nblk, 0))
        for j in range(ns)
    ]
    return pl.pallas_call(
        kfn,
        grid=(nblk,),
        in_specs=[pl.BlockSpec((Q, D), lambda i: (0, 0))] + kspecs + nspecs,
        out_specs=[
            pl.BlockSpec((Q, TOPK), lambda i: (0, 0)),
            pl.BlockSpec((Q, TOPK), lambda i: (0, 0)),
        ],
        out_shape=[
            jax.ShapeDtypeStruct((Q, TOPK), jnp.float32),
            jax.ShapeDtypeStruct((Q, TOPK), jnp.int32),
        ],
        scratch_shapes=[
            pltpu.VMEM((Q, ns * blk), jnp.float32),
            pltpu.VMEM((Q, TOPK), jnp.float32),
            pltpu.VMEM((Q, TOPK), jnp.int32),
        ],
        interpret=interpret,
    )(qn, *([keys] * ns), *([knorm] * ns))


@jax.jit
def kernel(queries, keys):
    qn = queries / (jnp.linalg.norm(queries, axis=-1, keepdims=True) + 1e-8)
    knorm = jnp.linalg.norm(keys, axis=-1, keepdims=True)
    K = keys.shape[0]
    ns = 4 if K % 4 == 0 else (2 if K % 2 == 0 else 1)
    seg = K // ns
    blk = next((b for b in (1000, 200, 40, 8) if seg % b == 0), seg)
    return _retrieve(qn, keys, knorm, ns, blk)
